# baseline (speedup 1.0000x reference)
import functools

import jax
import jax.numpy as jnp
from jax import lax
from jax.experimental import pallas as pl
from jax.experimental.pallas import tpu as pltpu
from jax.experimental.pallas import tpu_sc as plsc

NUM_EMB = 1000000
D = 128
C0, C1 = 20000, 200000
V2 = NUM_EMB - C1
_DNT = (((1,), (1,)), ((), ()))
_DN = (((1,), (0,)), ((), ()))


def _b_body(t2p, w2big, donor, out):
    res = lax.dot_general(t2p[...], w2big[...], _DN,
                          preferred_element_type=jnp.float32)
    out[...] = res.reshape(6400, D)


def _build_p_cluster2(t2p, w2big, donor):
    return pl.pallas_call(
        _b_body,
        grid=(125,),
        in_specs=[
            pl.BlockSpec((400, 128), lambda i: (i, 0)),
            pl.BlockSpec((128, 2048), lambda i: (0, 0)),
            pl.BlockSpec(memory_space=pl.ANY),
        ],
        out_specs=pl.BlockSpec((6400, D), lambda i: (i, 0)),
        out_shape=jax.ShapeDtypeStruct((NUM_EMB, D), jnp.float32),
        input_output_aliases={2: 0},
    )(t2p, w2big, donor)


def _a_body(t0, w0, t1, w1, out):
    i = pl.program_id(0)

    @pl.when(i < 5)
    def _():
        out[...] = lax.dot_general(t0[...], w0[...], _DNT,
                                   preferred_element_type=jnp.float32)

    @pl.when(i >= 5)
    def _():
        out[...] = lax.dot_general(t1[...].astype(jnp.bfloat16), w1[...],
                                   _DNT, preferred_element_type=jnp.float32)


def _build_p_cluster01(t0, w0, t1p, w1big):
    return pl.pallas_call(
        _a_body,
        grid=(50,),
        in_specs=[
            pl.BlockSpec((4000, 128), lambda i: (jnp.clip(i, 0, 4), 0)),
            pl.BlockSpec((D, 128), lambda i: (0, 0)),
            pl.BlockSpec((4000, 32), lambda i: (jnp.clip(i - 5, 0, 44), 0)),
            pl.BlockSpec((D, 32), lambda i: (0, 0)),
        ],
        out_specs=pl.BlockSpec((4000, D), lambda i: (200 + i, 0)),
        out_shape=jax.ShapeDtypeStruct((NUM_EMB, D), jnp.float32),
    )(t0, w0, t1p, w1big)


def _expand_weight(w, packs):
    eye = jnp.eye(packs, dtype=jnp.float32)
    big = jnp.einsum("ij,kd->ikjd", eye, w.T)
    return big.reshape(packs * w.shape[1], packs * D)


_info = plsc.get_sparse_core_info()
_NC, _NS = _info.num_cores, _info.num_subcores
_NW = _NC * _NS


def _make_gather(n_tokens):
    assert n_tokens % _NW == 0
    b_per_w = n_tokens // _NW
    chunk = 400
    assert b_per_w % (2 * chunk) == 0
    n_pairs = b_per_w // (2 * chunk)
    mesh = plsc.VectorSubcoreMesh(core_axis_name="c", subcore_axis_name="s")

    @functools.partial(
        pl.kernel,
        mesh=mesh,
        out_type=jax.ShapeDtypeStruct((n_tokens, D), jnp.float32),
        scratch_types=[
            pltpu.VMEM((chunk,), jnp.int32),
            pltpu.VMEM((chunk,), jnp.int32),
            pltpu.VMEM((chunk, D), jnp.float32),
            pltpu.VMEM((chunk, D), jnp.float32),
            pltpu.SemaphoreType.DMA,
            pltpu.SemaphoreType.DMA,
            pltpu.SemaphoreType.DMA,
            pltpu.SemaphoreType.DMA,
        ],
    )
    def gather_k(idx_hbm, p_hbm, out_hbm, idx_a, idx_b, rows_a, rows_b,
                 sg_a, sg_b, sw_a, sw_b):
        wid = lax.axis_index("s") * _NC + lax.axis_index("c")
        base = wid * b_per_w

        def load_idx(j, idx_v):
            pltpu.sync_copy(idx_hbm.at[pl.ds(base + j * chunk, chunk)], idx_v)

            def remap(k, carry):
                v = idx_v[pl.ds(k * 16, 16)] + V2
                idx_v[pl.ds(k * 16, 16)] = jnp.where(
                    v >= NUM_EMB, v - NUM_EMB, v)
                return carry

            lax.fori_loop(0, chunk // 16, remap, 0)

        load_idx(0, idx_a)
        pltpu.async_copy(p_hbm.at[idx_a], rows_a, sg_a)
        load_idx(1, idx_b)
        pltpu.async_copy(p_hbm.at[idx_b], rows_b, sg_b)

        def pair(k, carry):
            j0 = 2 * k
            off0 = base + j0 * chunk
            pltpu.make_async_copy(p_hbm.at[idx_a], rows_a, sg_a).wait()
            pltpu.async_copy(rows_a, out_hbm.at[pl.ds(off0, chunk)], sw_a)

            @pl.when(k < n_pairs - 1)
            def _():
                pltpu.make_async_copy(
                    rows_a, out_hbm.at[pl.ds(base, chunk)], sw_a).wait()
                load_idx(j0 + 2, idx_a)
                pltpu.async_copy(p_hbm.at[idx_a], rows_a, sg_a)

            pltpu.make_async_copy(p_hbm.at[idx_b], rows_b, sg_b).wait()
            pltpu.async_copy(rows_b, out_hbm.at[pl.ds(off0 + chunk, chunk)],
                             sw_b)

            @pl.when(k < n_pairs - 1)
            def _():
                pltpu.make_async_copy(
                    rows_b, out_hbm.at[pl.ds(base, chunk)], sw_b).wait()
                load_idx(j0 + 3, idx_b)
                pltpu.async_copy(p_hbm.at[idx_b], rows_b, sg_b)

            return carry

        lax.fori_loop(0, n_pairs, pair, 0)
        pltpu.make_async_copy(rows_a, out_hbm.at[pl.ds(base, chunk)],
                              sw_a).wait()
        pltpu.make_async_copy(rows_b, out_hbm.at[pl.ds(base, chunk)],
                              sw_b).wait()

    return gather_k


def kernel(emb_input, table0, W0, table1, W1, table2, W2):
    bsz, slen = emb_input.shape
    n = bsz * slen
    t2p = table2.reshape(50000, 128).astype(jnp.bfloat16)
    w2big = _expand_weight(W2, 16).astype(jnp.bfloat16)
    p01 = _build_p_cluster01(table0, W0, table1, W1.astype(jnp.bfloat16))
    proj = _build_p_cluster2(t2p, w2big, p01)
    flat = emb_input.reshape(n)
    out = _make_gather(n)(flat, proj)
    return out.reshape(bsz, slen, D)
